# baseline (speedup 1.0000x reference)
import functools

import jax
import jax.numpy as jnp
from jax import lax
from jax.experimental import pallas as pl
from jax.experimental.pallas import tpu as pltpu
from jax.experimental.pallas import tpu_sc as plsc

N = 10000
E = 320000
H = 128
NC = 2
NS = 16
NW = NC * NS
EPW = E // NW
LANES = 16
BN = 2000
NB = N // BN
BE = E // NB
UNROLL = 5
NITER = EPW // LANES
NP = 2048
PN = NB * NP


def _ab_body(x_ref, wu_ref, a2_ref, eat_hbm, we_ref, wm_ref, ei_ref,
             p_ref, t_ref, tm_ref, ei_out,
             ea2_ref, pscr_ref, tscr_ref, sem, sem0, sem1, tmax_ref):
    i = pl.program_id(0)

    @pl.when(i == 0)
    def _():
        pltpu.make_async_copy(eat_hbm.at[:, pl.ds(0, BE)], ea2_ref.at[0],
                              sem0).start()

    @pl.when((i + 1 < NB) & ((i + 1) % 2 == 0))
    def _():
        pltpu.make_async_copy(eat_hbm.at[:, pl.ds((i + 1) * BE, BE)],
                              ea2_ref.at[0], sem0).start()

    @pl.when((i + 1 < NB) & ((i + 1) % 2 == 1))
    def _():
        pltpu.make_async_copy(eat_hbm.at[:, pl.ds((i + 1) * BE, BE)],
                              ea2_ref.at[1], sem1).start()

    u = jnp.dot(x_ref[...], wu_ref[...], preferred_element_type=jnp.float32)
    hp = jnp.where(u > 0, u, 0.01 * u)
    pscr_ref[:, pl.ds(0, BN)] = lax.dot_general(
        a2_ref[...], hp, (((0,), (1,)), ((), ())),
        preferred_element_type=jnp.float32)

    wem = jnp.dot(we_ref[...], wm_ref[...], preferred_element_type=jnp.float32)

    @pl.when(i % 2 == 0)
    def _():
        pltpu.make_async_copy(eat_hbm.at[:, pl.ds(i * BE, BE)],
                              ea2_ref.at[0], sem0).wait()
        tscr_ref[...] = lax.dot_general(wem, ea2_ref[0],
                                        (((0,), (0,)), ((), ())),
                                        preferred_element_type=jnp.float32)

    @pl.when(i % 2 == 1)
    def _():
        pltpu.make_async_copy(eat_hbm.at[:, pl.ds(i * BE, BE)],
                              ea2_ref.at[1], sem1).wait()
        tscr_ref[...] = lax.dot_general(wem, ea2_ref[1],
                                        (((0,), (0,)), ((), ())),
                                        preferred_element_type=jnp.float32)

    sT = tscr_ref[...]
    bm = jnp.max(jnp.abs(sT))

    @pl.when(i == 0)
    def _():
        tmax_ref[0, 0] = bm

    @pl.when(i > 0)
    def _():
        tmax_ref[0, 0] = jnp.maximum(tmax_ref[0, 0], bm)

    cps = [
        pltpu.make_async_copy(pscr_ref.at[0], p_ref.at[pl.ds(i * NP, NP)],
                              sem),
        pltpu.make_async_copy(pscr_ref.at[1], p_ref.at[pl.ds(PN + i * NP, NP)],
                              sem),
        pltpu.make_async_copy(tscr_ref.at[0], t_ref.at[pl.ds(i * BE, BE)],
                              sem),
        pltpu.make_async_copy(ei_ref.at[0], ei_out.at[pl.ds(i * BE, BE)], sem),
        pltpu.make_async_copy(ei_ref.at[1],
                              ei_out.at[pl.ds(E + i * BE, BE)], sem),
    ]
    for cp in cps:
        cp.start()
    for cp in cps:
        cp.wait()

    @pl.when(i == pl.num_programs(0) - 1)
    def _():
        tm_ref[...] = jnp.full((1, H), tmax_ref[0, 0], dtype=jnp.float32)


def _ab_call(x, W_u, A2, eaT, W_e, W_m, edge_index):
    d_edge = W_e.shape[0]
    return pl.pallas_call(
        _ab_body,
        grid=(NB,),
        in_specs=[
            pl.BlockSpec((BN, H), lambda i: (i, 0)),
            pl.BlockSpec((H, H), lambda i: (0, 0)),
            pl.BlockSpec((H, 2), lambda i: (0, 0)),
            pl.BlockSpec(memory_space=pltpu.MemorySpace.HBM),
            pl.BlockSpec((d_edge, H), lambda i: (0, 0)),
            pl.BlockSpec((H, 1), lambda i: (0, 0)),
            pl.BlockSpec((2, BE), lambda i: (0, i)),
        ],
        out_specs=[
            pl.BlockSpec(memory_space=pltpu.MemorySpace.HBM),
            pl.BlockSpec(memory_space=pltpu.MemorySpace.HBM),
            pl.BlockSpec((1, H), lambda i: (0, 0)),
            pl.BlockSpec(memory_space=pltpu.MemorySpace.HBM),
        ],
        out_shape=[
            jax.ShapeDtypeStruct((2 * PN,), jnp.float32),
            jax.ShapeDtypeStruct((E,), jnp.float32),
            jax.ShapeDtypeStruct((1, H), jnp.float32),
            jax.ShapeDtypeStruct((2 * E,), jnp.int32),
        ],
        scratch_shapes=[pltpu.VMEM((2, 16, BE), jnp.float32),
                        pltpu.VMEM((2, NP), jnp.float32),
                        pltpu.VMEM((1, BE), jnp.float32),
                        pltpu.SemaphoreType.DMA,
                        pltpu.SemaphoreType.DMA,
                        pltpu.SemaphoreType.DMA,
                        pltpu.SMEM((1, 1), jnp.float32)],
    )(x, W_u, A2, eaT, W_e, W_m, edge_index)


def _sc_edge_body(ei_hbm, t_hbm, p_hbm, tm_hbm, w_out,
                  src_v, dst_v, t_v, p1_v, p2_v, w_v, tm_v, sem):
    wid = lax.axis_index("s") * NC + lax.axis_index("c")
    base = wid * EPW
    cps = [
        pltpu.make_async_copy(ei_hbm.at[pl.ds(base, EPW)], src_v, sem),
        pltpu.make_async_copy(ei_hbm.at[pl.ds(E + base, EPW)], dst_v, sem),
        pltpu.make_async_copy(t_hbm.at[pl.ds(base, EPW)], t_v, sem),
        pltpu.make_async_copy(tm_hbm, tm_v, sem),
    ] + [
        pltpu.make_async_copy(p_hbm.at[pl.ds(b * NP, BN)],
                              p1_v.at[pl.ds(b * BN, BN)], sem)
        for b in range(NB)
    ] + [
        pltpu.make_async_copy(p_hbm.at[pl.ds(PN + b * NP, BN)],
                              p2_v.at[pl.ds(b * BN, BN)], sem)
        for b in range(NB)
    ]
    for cp in cps:
        cp.start()

    def bzero(i, carry):
        w_v[pl.ds(pl.multiple_of(i * LANES, LANES), LANES)] = (
            jnp.zeros((LANES,), jnp.float32))
        return carry

    lax.fori_loop(0, N // LANES, bzero, 0)
    for cp in cps:
        cp.wait()
    cc = jnp.max(tm_v[pl.ds(0, LANES)])

    def body(i, carry):
        sls, sis, scs = [], [], []
        for j in range(UNROLL):
            sl = pl.ds(pl.multiple_of((i * UNROLL + j) * LANES, LANES), LANES)
            si = src_v[sl]
            di = dst_v[sl]
            sc = plsc.load_gather(p1_v, [si]) + plsc.load_gather(p2_v, [di])
            sls.append(sl)
            sis.append(si)
            scs.append(sc)
        es = [jnp.exp(-jnp.abs(sc)) for sc in scs]
        zs = []
        for j in range(UNROLL):
            d = 1.0 + es[j]
            r = 1.4117647 - 0.4705882 * d
            r = r * (2.0 - d * r)
            r = r * (2.0 - d * r)
            sig = jnp.where(scs[j] >= 0, r, 1.0 - r)
            zs.append(sig * t_v[sls[j]] - cc)
        exs = [jnp.exp(z) for z in zs]
        for j in range(UNROLL):
            plsc.addupdate_scatter(w_v, [sis[j]], exs[j])
        return carry

    lax.fori_loop(0, NITER // UNROLL, body, 0)

    for b in range(NB):
        pltpu.sync_copy(w_v.at[pl.ds(b * BN, BN)],
                        w_out.at[b, wid, pl.ds(0, BN)])


def _sc_call(ei_flat, t, p, tm):
    mesh = plsc.VectorSubcoreMesh(core_axis_name="c", subcore_axis_name="s")
    f = functools.partial(
        pl.kernel,
        mesh=mesh,
        compiler_params=pltpu.CompilerParams(
            needs_layout_passes=False, use_tc_tiling_on_sc=False),
        out_type=jax.ShapeDtypeStruct((NB, NW, NP), jnp.float32),
        scratch_types=[
            pltpu.VMEM((EPW,), jnp.int32),
            pltpu.VMEM((EPW,), jnp.int32),
            pltpu.VMEM((EPW,), jnp.float32),
            pltpu.VMEM((N,), jnp.float32),
            pltpu.VMEM((N,), jnp.float32),
            pltpu.VMEM((N,), jnp.float32),
            pltpu.VMEM((H,), jnp.float32),
            pltpu.SemaphoreType.DMA,
        ],
    )(_sc_edge_body)
    return f(ei_flat, t, p, tm)


def _final_body(w_ref, x_ref, wu_ref, w1, b1, w2, b2, w3, b3, w4, b4,
                out_ref, wscr_ref, sem, acc_ref, accs_ref):
    i = pl.program_id(0)

    @pl.when(i == 0)
    def _init():
        acc_ref[...] = jnp.zeros_like(acc_ref)
        accs_ref[0, 0] = 0.0

    cp = pltpu.make_async_copy(w_ref.at[i], wscr_ref, sem)
    cp.start()
    u = jnp.dot(x_ref[...], wu_ref[...], preferred_element_type=jnp.float32)
    g = jnp.where(u > 0, u, 0.0001 * u)
    cp.wait()
    wblk = wscr_ref[:, pl.ds(0, BN)]
    cw = jnp.sum(wblk, axis=0, keepdims=True)
    acc_ref[...] += jnp.dot(cw, g, preferred_element_type=jnp.float32)
    accs_ref[0, 0] += jnp.sum(cw)

    @pl.when(i == pl.num_programs(0) - 1)
    def _finish():
        hg = acc_ref[...] / accs_ref[0, 0]
        o = jnp.dot(hg, w1[...], preferred_element_type=jnp.float32) + b1[...]
        o = jnp.maximum(o, 0.0)
        o = jnp.dot(o, w2[...], preferred_element_type=jnp.float32) + b2[...]
        o = jnp.maximum(o, 0.0)
        o = jnp.dot(o, w3[...], preferred_element_type=jnp.float32) + b3[...]
        o = jnp.maximum(o, 0.0)
        out_ref[...] = (jnp.dot(o, w4[...], preferred_element_type=jnp.float32)
                        + b4[...])


def _final_call(w3d, x, W_u, W1, b1, W2, b2, W3, b3, W4, b4):
    full = lambda i: (0, 0)
    return pl.pallas_call(
        _final_body,
        grid=(NB,),
        in_specs=[
            pl.BlockSpec(memory_space=pltpu.MemorySpace.HBM),
            pl.BlockSpec((BN, H), lambda i: (i, 0)),
            pl.BlockSpec((H, H), full),
            pl.BlockSpec(W1.shape, full),
            pl.BlockSpec(b1.shape, full),
            pl.BlockSpec(W2.shape, full),
            pl.BlockSpec(b2.shape, full),
            pl.BlockSpec(W3.shape, full),
            pl.BlockSpec(b3.shape, full),
            pl.BlockSpec(W4.shape, full),
            pl.BlockSpec(b4.shape, full),
        ],
        out_specs=pl.BlockSpec((1, 1), full),
        out_shape=jax.ShapeDtypeStruct((1, 1), jnp.float32),
        scratch_shapes=[
            pltpu.VMEM((NW, NP), jnp.float32),
            pltpu.SemaphoreType.DMA,
            pltpu.VMEM((1, H), jnp.float32),
            pltpu.SMEM((1, 1), jnp.float32),
        ],
    )(w3d, x, W_u, W1, b1, W2, b2, W3, b3, W4, b4)


def kernel(x, edge_index, edge_attr, W_u, a, W_e, W_m,
           W1, b1, W2, b2, W3, b3, W4, b4):
    A2 = jnp.concatenate([a[:H], a[H:]], axis=1)

    p, t, tm, ei = _ab_call(x, W_u, A2, edge_attr.T, W_e, W_m, edge_index)

    w3d = _sc_call(ei, t, p, tm.reshape(H))

    return _final_call(w3d, x, W_u,
                       W1, b1.reshape(1, -1), W2, b2.reshape(1, -1),
                       W3, b3.reshape(1, -1), W4, b4.reshape(1, -1))
